# baseline (speedup 1.0000x reference)
import functools

import jax
import jax.numpy as jnp
from jax.experimental import pallas as pl
from jax.experimental.pallas import tpu as pltpu
from jax.experimental.pallas import tpu_sc as plsc

_N = 8192
_D = 256
_R = 512
_C = 1024
_NI = _N // _R
_NJ = _N // _C
_RS = 64
_LK = 128


def _phase1_body(x2_ref, e_ref, xsq_ref, esq_ref, rv_ref, rc_ref):
    j = pl.program_id(1)

    @pl.when(j == 0)
    def _():
        rv_ref[...] = jnp.full((_R, _LK), jnp.float32(3.0e38), jnp.float32)
        rc_ref[...] = jnp.zeros((_R, _LK), jnp.int32)

    xe2 = jax.lax.dot_general(x2_ref[...], e_ref[...], (((1,), (1,)), ((), ())),
                              preferred_element_type=jnp.float32)

    for rs in range(_R // _RS):
        rsl = pl.ds(rs * _RS, _RS)
        xsq_s = xsq_ref[rsl, :]
        tv = None
        tc = None
        for c in range(_C // _LK):
            esq_c = esq_ref[:, pl.ds(c * _LK, _LK)]
            t1 = xsq_s + esq_c
            dch = t1 - xe2[rs * _RS:(rs + 1) * _RS, c * _LK:(c + 1) * _LK]
            if tv is None:
                tv = dch
                tc = jnp.zeros((_RS, _LK), jnp.int32)
            else:
                b = dch < tv
                tv = jnp.where(b, dch, tv)
                tc = jnp.where(b, jnp.int32(c), tc)
        old = rv_ref[rsl, :]
        b = tv < old
        rv_ref[rsl, :] = jnp.where(b, tv, old)
        rc_ref[rsl, :] = jnp.where(b, tc + jnp.int32(j * (_C // _LK)),
                                   rc_ref[rsl, :])


def _make_phase1(interpret=False):
    return pl.pallas_call(
        _phase1_body,
        grid=(_NI, _NJ),
        in_specs=[
            pl.BlockSpec((_R, _D), lambda i, j: (i, 0)),
            pl.BlockSpec((_C, _D), lambda i, j: (j, 0)),
            pl.BlockSpec((_R, 1), lambda i, j: (i, 0)),
            pl.BlockSpec((1, _C), lambda i, j: (0, j)),
        ],
        out_specs=[
            pl.BlockSpec((_R, _LK), lambda i, j: (i, 0)),
            pl.BlockSpec((_R, _LK), lambda i, j: (i, 0)),
        ],
        out_shape=[
            jax.ShapeDtypeStruct((_N, _LK), jnp.float32),
            jax.ShapeDtypeStruct((_N, _LK), jnp.int32),
        ],
        compiler_params=pltpu.CompilerParams(
            dimension_semantics=("arbitrary", "arbitrary")),
        interpret=interpret,
    )


def _gram_body(e_ref, esq_ref, ortho_ref, gram, sum_e4):
    t = pl.program_id(0)
    g = jax.lax.dot_general(e_ref[...], e_ref[...], (((0,), (0,)), ((), ())),
                            preferred_element_type=jnp.float32)
    esq_v = esq_ref[...]
    s4 = jnp.sum(esq_v * esq_v)

    @pl.when(t == 0)
    def _():
        gram[...] = g
        sum_e4[0, 0] = s4

    @pl.when(t > 0)
    def _():
        gram[...] += g
        sum_e4[0, 0] += s4

    @pl.when(t == _NJ - 1)
    def _():
        m = gram[...]
        ortho_sq = jnp.sum(m * m) - sum_e4[0, 0]
        ortho_ref[...] = jnp.full((1, 1), jnp.sqrt(jnp.maximum(ortho_sq, 0.0)),
                                  jnp.float32)


def _make_gram(interpret=False):
    return pl.pallas_call(
        _gram_body,
        grid=(_NJ,),
        in_specs=[
            pl.BlockSpec((_C, _D), lambda t: (t, 0)),
            pl.BlockSpec((1, _C), lambda t: (0, t)),
        ],
        out_specs=pl.BlockSpec((1, 1), lambda t: (0, 0)),
        out_shape=jax.ShapeDtypeStruct((1, 1), jnp.float32),
        scratch_shapes=[
            pltpu.VMEM((_D, _D), jnp.float32),
            pltpu.SMEM((1, 1), jnp.float32),
        ],
        compiler_params=pltpu.CompilerParams(
            dimension_semantics=("arbitrary",)),
        interpret=interpret,
    )


def _fin_body(rv_ref, rc_ref, ortho_ref, idx_ref, loss_ref, sum_min):
    i = pl.program_id(0)
    lane = jax.lax.broadcasted_iota(jnp.int32, (_RS, _LK), 1)
    acc = jnp.float32(0.0)
    for rs in range(_R // _RS):
        rsl = pl.ds(rs * _RS, _RS)
        v = rv_ref[rsl, :]
        tmin = jnp.min(v, axis=1, keepdims=True)
        col = rc_ref[rsl, :] * _LK + lane
        targ = jnp.min(jnp.where(v == tmin, col, jnp.int32(2 ** 30)),
                       axis=1, keepdims=True)
        idx_ref[rsl, :] = targ
        acc = acc + jnp.sum(tmin)

    @pl.when(i == 0)
    def _():
        sum_min[0, 0] = acc

    @pl.when(i > 0)
    def _():
        sum_min[0, 0] += acc

    @pl.when(i == _NI - 1)
    def _():
        mse = sum_min[0, 0] / jnp.float32(_N * _D)
        loss_ref[...] = jnp.full(
            (1, 1), mse + 0.25 * mse + 0.09 * ortho_ref[0, 0], jnp.float32)


def _make_fin(interpret=False):
    return pl.pallas_call(
        _fin_body,
        grid=(_NI,),
        in_specs=[
            pl.BlockSpec((_R, _LK), lambda i: (i, 0)),
            pl.BlockSpec((_R, _LK), lambda i: (i, 0)),
            pl.BlockSpec((1, 1), lambda i: (0, 0)),
        ],
        out_specs=[
            pl.BlockSpec((_R, 1), lambda i: (i, 0)),
            pl.BlockSpec((1, 1), lambda i: (0, 0)),
        ],
        out_shape=[
            jax.ShapeDtypeStruct((_N, 1), jnp.int32),
            jax.ShapeDtypeStruct((1, 1), jnp.float32),
        ],
        scratch_shapes=[
            pltpu.SMEM((1, 1), jnp.float32),
        ],
        compiler_params=pltpu.CompilerParams(
            dimension_semantics=("arbitrary",)),
        interpret=interpret,
    )


def _ew_body(x_ref, q_ref, o_ref):
    o_ref[...] = x_ref[...] + (q_ref[...] - x_ref[...])


def _make_ew(interpret=False):
    return pl.pallas_call(
        _ew_body,
        grid=(8,),
        in_specs=[
            pl.BlockSpec((1024, _D), lambda i: (i, 0)),
            pl.BlockSpec((1024, _D), lambda i: (i, 0)),
        ],
        out_specs=pl.BlockSpec((1024, _D), lambda i: (i, 0)),
        out_shape=jax.ShapeDtypeStruct((_N, _D), jnp.float32),
        interpret=interpret,
    )


def _sc_gather(table, idx):
    mesh = plsc.VectorSubcoreMesh(core_axis_name="c", subcore_axis_name="s")
    n_workers = 32
    bpw = _N // n_workers

    @functools.partial(
        pl.kernel,
        out_type=jax.ShapeDtypeStruct((_N, _D), jnp.float32),
        mesh=mesh,
        scratch_types=[
            pltpu.VMEM((bpw,), jnp.int32),
            pltpu.VMEM((bpw, _D), jnp.float32),
            pltpu.SemaphoreType.DMA,
        ],
    )
    def gather_kernel(table_hbm, idx_hbm, out_hbm, idx_v, rows_v, sem):
        wid = jax.lax.axis_index("s") * 2 + jax.lax.axis_index("c")
        base = wid * bpw
        pltpu.sync_copy(idx_hbm.at[pl.ds(base, bpw)], idx_v)
        pltpu.async_copy(table_hbm.at[idx_v], rows_v, sem).wait()
        pltpu.sync_copy(rows_v, out_hbm.at[pl.ds(base, bpw)])

    return gather_kernel(table, idx)


def kernel(inputs, embedding_weight):
    input_shape = inputs.shape
    x = inputs.reshape(-1, _D)
    xsq = jnp.sum(x ** 2, axis=1, keepdims=True)
    esq = jnp.sum(embedding_weight ** 2, axis=1)
    esq2 = esq.reshape(1, _N)

    rv, rc = _make_phase1()(2.0 * x, embedding_weight, xsq, esq2)
    ortho11 = _make_gram()(embedding_weight, esq2)
    idx2d, loss11 = _make_fin()(rv, rc, ortho11)
    q = _sc_gather(embedding_weight, idx2d.reshape(_N))
    quantized_st = _make_ew()(x, q)
    return (quantized_st.reshape(input_shape), loss11[0, 0], idx2d, inputs)


# speedup vs baseline: 1.0727x; 1.0727x over previous
import functools

import jax
import jax.numpy as jnp
from jax.experimental import pallas as pl
from jax.experimental.pallas import tpu as pltpu
from jax.experimental.pallas import tpu_sc as plsc

_N = 8192
_D = 256
_R = 512
_C = 1024
_NI = _N // _R
_NJ = _N // _C


_RS = 64
_LK = 128


def _phase1_body(x2_ref, e_ref, xsq_ref, esq_ref, idx_ref, loss_ref,
                 rv, rc, gram, sum_min, sum_e4):
    i = pl.program_id(0)
    j = pl.program_id(1)

    @pl.when(j == 0)
    def _():
        rv[...] = jnp.full((_R, _LK), jnp.float32(3.0e38), jnp.float32)
        rc[...] = jnp.zeros((_R, _LK), jnp.int32)

    xe2 = jax.lax.dot_general(x2_ref[...], e_ref[...], (((1,), (1,)), ((), ())),
                              preferred_element_type=jnp.float32)

    for rs in range(_R // _RS):
        rsl = pl.ds(rs * _RS, _RS)
        xsq_s = xsq_ref[rsl, :]
        tv = None
        tc = None
        for c in range(_C // _LK):
            esq_c = esq_ref[:, pl.ds(c * _LK, _LK)]
            t1 = xsq_s + esq_c
            dch = t1 - xe2[rs * _RS:(rs + 1) * _RS, c * _LK:(c + 1) * _LK]
            if tv is None:
                tv = dch
                tc = jnp.zeros((_RS, _LK), jnp.int32)
            else:
                b = dch < tv
                tv = jnp.where(b, dch, tv)
                tc = jnp.where(b, jnp.int32(c), tc)
        old = rv[rsl, :]
        b = tv < old
        rv[rsl, :] = jnp.where(b, tv, old)
        rc[rsl, :] = jnp.where(b, tc + jnp.int32(j * (_C // _LK)), rc[rsl, :])

    @pl.when(i == 0)
    def _():
        g = jax.lax.dot_general(e_ref[...], e_ref[...], (((0,), (0,)), ((), ())),
                                preferred_element_type=jnp.float32)
        esq_v = esq_ref[...]
        s4 = jnp.sum(esq_v * esq_v)

        @pl.when(j == 0)
        def _():
            gram[...] = g
            sum_e4[0, 0] = s4

        @pl.when(j > 0)
        def _():
            gram[...] += g
            sum_e4[0, 0] += s4

    @pl.when(j == _NJ - 1)
    def _():
        lane = jax.lax.broadcasted_iota(jnp.int32, (_RS, _LK), 1)
        acc = jnp.float32(0.0)
        for rs in range(_R // _RS):
            rsl = pl.ds(rs * _RS, _RS)
            v = rv[rsl, :]
            tmin = jnp.min(v, axis=1, keepdims=True)
            col = rc[rsl, :] * _LK + lane
            targ = jnp.min(jnp.where(v == tmin, col, jnp.int32(2 ** 30)),
                           axis=1, keepdims=True)
            idx_ref[rsl, :] = targ
            acc = acc + jnp.sum(tmin)

        @pl.when(i == 0)
        def _():
            sum_min[0, 0] = acc

        @pl.when(i > 0)
        def _():
            sum_min[0, 0] += acc

    @pl.when((i == _NI - 1) & (j == _NJ - 1))
    def _():
        m = gram[...]
        ortho_sq = jnp.sum(m * m) - sum_e4[0, 0]
        ortho = jnp.sqrt(jnp.maximum(ortho_sq, 0.0))
        mse = sum_min[0, 0] / jnp.float32(_N * _D)
        loss_ref[...] = jnp.full((1, 1), mse + 0.25 * mse + 0.09 * ortho,
                                 jnp.float32)


def _make_phase1(interpret=False):
    return pl.pallas_call(
        _phase1_body,
        grid=(_NI, _NJ),
        in_specs=[
            pl.BlockSpec((_R, _D), lambda i, j: (i, 0)),
            pl.BlockSpec((_C, _D), lambda i, j: (j, 0)),
            pl.BlockSpec((_R, 1), lambda i, j: (i, 0)),
            pl.BlockSpec((1, _C), lambda i, j: (0, j)),
        ],
        out_specs=[
            pl.BlockSpec((_R, 1), lambda i, j: (i, 0)),
            pl.BlockSpec((1, 1), lambda i, j: (0, 0)),
        ],
        out_shape=[
            jax.ShapeDtypeStruct((_N, 1), jnp.int32),
            jax.ShapeDtypeStruct((1, 1), jnp.float32),
        ],
        scratch_shapes=[
            pltpu.VMEM((_R, _LK), jnp.float32),
            pltpu.VMEM((_R, _LK), jnp.int32),
            pltpu.VMEM((_D, _D), jnp.float32),
            pltpu.SMEM((1, 1), jnp.float32),
            pltpu.SMEM((1, 1), jnp.float32),
        ],
        compiler_params=pltpu.CompilerParams(
            dimension_semantics=("arbitrary", "arbitrary")),
        interpret=interpret,
    )


def _ew_body(x_ref, q_ref, o_ref):
    o_ref[...] = x_ref[...] + (q_ref[...] - x_ref[...])


def _make_ew(interpret=False):
    return pl.pallas_call(
        _ew_body,
        grid=(8,),
        in_specs=[
            pl.BlockSpec((1024, _D), lambda i: (i, 0)),
            pl.BlockSpec((1024, _D), lambda i: (i, 0)),
        ],
        out_specs=pl.BlockSpec((1024, _D), lambda i: (i, 0)),
        out_shape=jax.ShapeDtypeStruct((_N, _D), jnp.float32),
        interpret=interpret,
    )


def _sc_gather(table, idx):
    mesh = plsc.VectorSubcoreMesh(core_axis_name="c", subcore_axis_name="s")
    n_workers = 32
    bpw = _N // n_workers

    @functools.partial(
        pl.kernel,
        out_type=jax.ShapeDtypeStruct((_N, _D), jnp.float32),
        mesh=mesh,
        scratch_types=[
            pltpu.VMEM((bpw,), jnp.int32),
            pltpu.VMEM((bpw, _D), jnp.float32),
            pltpu.SemaphoreType.DMA,
        ],
    )
    def gather_kernel(table_hbm, idx_hbm, out_hbm, idx_v, rows_v, sem):
        wid = jax.lax.axis_index("s") * 2 + jax.lax.axis_index("c")
        base = wid * bpw
        pltpu.sync_copy(idx_hbm.at[pl.ds(base, bpw)], idx_v)
        pltpu.async_copy(table_hbm.at[idx_v], rows_v, sem).wait()
        pltpu.sync_copy(rows_v, out_hbm.at[pl.ds(base, bpw)])

    return gather_kernel(table, idx)


def kernel(inputs, embedding_weight):
    input_shape = inputs.shape
    x = inputs.reshape(-1, _D)
    xsq = jnp.sum(x ** 2, axis=1, keepdims=True)
    esq = jnp.sum(embedding_weight ** 2, axis=1)

    idx2d, loss11 = _make_phase1()(2.0 * x, embedding_weight, xsq,
                                   esq.reshape(1, _N))
    q = _sc_gather(embedding_weight, idx2d.reshape(_N))
    quantized_st = _make_ew()(x, q)
    return (quantized_st.reshape(input_shape), loss11[0, 0], idx2d, inputs)


# speedup vs baseline: 1.4315x; 1.3345x over previous
import functools

import jax
import jax.numpy as jnp
from jax.experimental import pallas as pl
from jax.experimental.pallas import tpu as pltpu
from jax.experimental.pallas import tpu_sc as plsc

_N = 8192
_D = 256
_R = 2048
_C = 1024
_NI = _N // _R
_NJ = _N // _C


_RS = 64
_LK = 128


def _phase1_body(x2_ref, e_ref, xsq_ref, esq_ref, idx_ref, loss_ref,
                 rv, rc, gram, sum_min, sum_e4):
    i = pl.program_id(0)
    j = pl.program_id(1)

    @pl.when(j == 0)
    def _():
        rv[...] = jnp.full((_R, _LK), jnp.float32(3.0e38), jnp.float32)
        rc[...] = jnp.zeros((_R, _LK), jnp.int32)

    xe2 = jax.lax.dot_general(x2_ref[...], e_ref[...], (((1,), (1,)), ((), ())),
                              preferred_element_type=jnp.float32)

    for rs in range(_R // _RS):
        rsl = pl.ds(rs * _RS, _RS)
        xsq_s = xsq_ref[rsl, :]
        tv = None
        tc = None
        for c in range(_C // _LK):
            esq_c = esq_ref[:, pl.ds(c * _LK, _LK)]
            t1 = xsq_s + esq_c
            dch = t1 - xe2[rs * _RS:(rs + 1) * _RS, c * _LK:(c + 1) * _LK]
            if tv is None:
                tv = dch
                tc = jnp.zeros((_RS, _LK), jnp.int32)
            else:
                b = dch < tv
                tv = jnp.where(b, dch, tv)
                tc = jnp.where(b, jnp.int32(c), tc)
        old = rv[rsl, :]
        b = tv < old
        rv[rsl, :] = jnp.where(b, tv, old)
        rc[rsl, :] = jnp.where(b, tc + jnp.int32(j * (_C // _LK)), rc[rsl, :])

    @pl.when(i == 0)
    def _():
        g = jax.lax.dot_general(e_ref[...], e_ref[...], (((0,), (0,)), ((), ())),
                                preferred_element_type=jnp.float32)
        esq_v = esq_ref[...]
        s4 = jnp.sum(esq_v * esq_v)

        @pl.when(j == 0)
        def _():
            gram[...] = g
            sum_e4[0, 0] = s4

        @pl.when(j > 0)
        def _():
            gram[...] += g
            sum_e4[0, 0] += s4

    @pl.when(j == _NJ - 1)
    def _():
        lane = jax.lax.broadcasted_iota(jnp.int32, (_RS, _LK), 1)
        acc = jnp.float32(0.0)
        for rs in range(_R // _RS):
            rsl = pl.ds(rs * _RS, _RS)
            v = rv[rsl, :]
            tmin = jnp.min(v, axis=1, keepdims=True)
            col = rc[rsl, :] * _LK + lane
            targ = jnp.min(jnp.where(v == tmin, col, jnp.int32(2 ** 30)),
                           axis=1, keepdims=True)
            idx_ref[rsl, :] = targ
            acc = acc + jnp.sum(tmin)

        @pl.when(i == 0)
        def _():
            sum_min[0, 0] = acc

        @pl.when(i > 0)
        def _():
            sum_min[0, 0] += acc

    @pl.when((i == _NI - 1) & (j == _NJ - 1))
    def _():
        m = gram[...]
        ortho_sq = jnp.sum(m * m) - sum_e4[0, 0]
        ortho = jnp.sqrt(jnp.maximum(ortho_sq, 0.0))
        mse = sum_min[0, 0] / jnp.float32(_N * _D)
        loss_ref[...] = jnp.full((1, 1), mse + 0.25 * mse + 0.09 * ortho,
                                 jnp.float32)


def _make_phase1(interpret=False):
    return pl.pallas_call(
        _phase1_body,
        grid=(_NI, _NJ),
        in_specs=[
            pl.BlockSpec((_R, _D), lambda i, j: (i, 0)),
            pl.BlockSpec((_C, _D), lambda i, j: (j, 0)),
            pl.BlockSpec((_R, 1), lambda i, j: (i, 0)),
            pl.BlockSpec((1, _C), lambda i, j: (0, j)),
        ],
        out_specs=[
            pl.BlockSpec((_R, 1), lambda i, j: (i, 0)),
            pl.BlockSpec((1, 1), lambda i, j: (0, 0)),
        ],
        out_shape=[
            jax.ShapeDtypeStruct((_N, 1), jnp.int32),
            jax.ShapeDtypeStruct((1, 1), jnp.float32),
        ],
        scratch_shapes=[
            pltpu.VMEM((_R, _LK), jnp.float32),
            pltpu.VMEM((_R, _LK), jnp.int32),
            pltpu.VMEM((_D, _D), jnp.float32),
            pltpu.SMEM((1, 1), jnp.float32),
            pltpu.SMEM((1, 1), jnp.float32),
        ],
        compiler_params=pltpu.CompilerParams(
            dimension_semantics=("arbitrary", "arbitrary")),
        interpret=interpret,
    )


def _ew_body(x_ref, q_ref, o_ref):
    o_ref[...] = x_ref[...] + (q_ref[...] - x_ref[...])


def _make_ew(interpret=False):
    return pl.pallas_call(
        _ew_body,
        grid=(8,),
        in_specs=[
            pl.BlockSpec((1024, _D), lambda i: (i, 0)),
            pl.BlockSpec((1024, _D), lambda i: (i, 0)),
        ],
        out_specs=pl.BlockSpec((1024, _D), lambda i: (i, 0)),
        out_shape=jax.ShapeDtypeStruct((_N, _D), jnp.float32),
        interpret=interpret,
    )


def _sc_gather(table, idx):
    mesh = plsc.VectorSubcoreMesh(core_axis_name="c", subcore_axis_name="s")
    n_workers = 32
    bpw = _N // n_workers

    @functools.partial(
        pl.kernel,
        out_type=jax.ShapeDtypeStruct((_N, _D), jnp.float32),
        mesh=mesh,
        scratch_types=[
            pltpu.VMEM((bpw,), jnp.int32),
            pltpu.VMEM((bpw, _D), jnp.float32),
            pltpu.SemaphoreType.DMA,
        ],
    )
    def gather_kernel(table_hbm, idx_hbm, out_hbm, idx_v, rows_v, sem):
        wid = jax.lax.axis_index("s") * 2 + jax.lax.axis_index("c")
        base = wid * bpw
        pltpu.sync_copy(idx_hbm.at[pl.ds(base, bpw)], idx_v)
        pltpu.async_copy(table_hbm.at[idx_v], rows_v, sem).wait()
        pltpu.sync_copy(rows_v, out_hbm.at[pl.ds(base, bpw)])

    return gather_kernel(table, idx)


def kernel(inputs, embedding_weight):
    input_shape = inputs.shape
    x = inputs.reshape(-1, _D)
    xsq = jnp.sum(x ** 2, axis=1, keepdims=True)
    esq = jnp.sum(embedding_weight ** 2, axis=1)

    idx2d, loss11 = _make_phase1()(2.0 * x, embedding_weight, xsq,
                                   esq.reshape(1, _N))
    q = _sc_gather(embedding_weight, idx2d.reshape(_N))
    quantized_st = _make_ew()(x, q)
    return (quantized_st.reshape(input_shape), loss11[0, 0], idx2d, inputs)


# speedup vs baseline: 1.4561x; 1.0172x over previous
import functools

import jax
import jax.numpy as jnp
from jax.experimental import pallas as pl
from jax.experimental.pallas import tpu as pltpu
from jax.experimental.pallas import tpu_sc as plsc

_N = 8192
_D = 256
_R = 4096
_C = 1024
_NI = _N // _R
_NJ = _N // _C


_RS = 64
_LK = 128


def _phase1_body(x2_ref, e_ref, xsq_ref, esq_ref, idx_ref, loss_ref,
                 rv, rc, gram, sum_min, sum_e4):
    i = pl.program_id(0)
    j = pl.program_id(1)

    @pl.when(j == 0)
    def _():
        rv[...] = jnp.full((_R, _LK), jnp.float32(3.0e38), jnp.float32)
        rc[...] = jnp.zeros((_R, _LK), jnp.int32)

    xe2 = jax.lax.dot_general(x2_ref[...], e_ref[...], (((1,), (1,)), ((), ())),
                              preferred_element_type=jnp.float32)

    for rs in range(_R // _RS):
        rsl = pl.ds(rs * _RS, _RS)
        xsq_s = xsq_ref[rsl, :]
        tv = None
        tc = None
        for c in range(_C // _LK):
            esq_c = esq_ref[:, pl.ds(c * _LK, _LK)]
            t1 = xsq_s + esq_c
            dch = t1 - xe2[rs * _RS:(rs + 1) * _RS, c * _LK:(c + 1) * _LK]
            if tv is None:
                tv = dch
                tc = jnp.zeros((_RS, _LK), jnp.int32)
            else:
                b = dch < tv
                tv = jnp.where(b, dch, tv)
                tc = jnp.where(b, jnp.int32(c), tc)
        old = rv[rsl, :]
        b = tv < old
        rv[rsl, :] = jnp.where(b, tv, old)
        rc[rsl, :] = jnp.where(b, tc + jnp.int32(j * (_C // _LK)), rc[rsl, :])

    @pl.when(i == 0)
    def _():
        g = jax.lax.dot_general(e_ref[...], e_ref[...], (((0,), (0,)), ((), ())),
                                preferred_element_type=jnp.float32)
        esq_v = esq_ref[...]
        s4 = jnp.sum(esq_v * esq_v)

        @pl.when(j == 0)
        def _():
            gram[...] = g
            sum_e4[0, 0] = s4

        @pl.when(j > 0)
        def _():
            gram[...] += g
            sum_e4[0, 0] += s4

    @pl.when(j == _NJ - 1)
    def _():
        lane = jax.lax.broadcasted_iota(jnp.int32, (_RS, _LK), 1)
        acc = jnp.float32(0.0)
        for rs in range(_R // _RS):
            rsl = pl.ds(rs * _RS, _RS)
            v = rv[rsl, :]
            tmin = jnp.min(v, axis=1, keepdims=True)
            col = rc[rsl, :] * _LK + lane
            targ = jnp.min(jnp.where(v == tmin, col, jnp.int32(2 ** 30)),
                           axis=1, keepdims=True)
            idx_ref[rsl, :] = targ
            acc = acc + jnp.sum(tmin)

        @pl.when(i == 0)
        def _():
            sum_min[0, 0] = acc

        @pl.when(i > 0)
        def _():
            sum_min[0, 0] += acc

    @pl.when((i == _NI - 1) & (j == _NJ - 1))
    def _():
        m = gram[...]
        ortho_sq = jnp.sum(m * m) - sum_e4[0, 0]
        ortho = jnp.sqrt(jnp.maximum(ortho_sq, 0.0))
        mse = sum_min[0, 0] / jnp.float32(_N * _D)
        loss_ref[...] = jnp.full((1, 1), mse + 0.25 * mse + 0.09 * ortho,
                                 jnp.float32)


def _make_phase1(interpret=False):
    return pl.pallas_call(
        _phase1_body,
        grid=(_NI, _NJ),
        in_specs=[
            pl.BlockSpec((_R, _D), lambda i, j: (i, 0)),
            pl.BlockSpec((_C, _D), lambda i, j: (j, 0)),
            pl.BlockSpec((_R, 1), lambda i, j: (i, 0)),
            pl.BlockSpec((1, _C), lambda i, j: (0, j)),
        ],
        out_specs=[
            pl.BlockSpec((_R, 1), lambda i, j: (i, 0)),
            pl.BlockSpec((1, 1), lambda i, j: (0, 0)),
        ],
        out_shape=[
            jax.ShapeDtypeStruct((_N, 1), jnp.int32),
            jax.ShapeDtypeStruct((1, 1), jnp.float32),
        ],
        scratch_shapes=[
            pltpu.VMEM((_R, _LK), jnp.float32),
            pltpu.VMEM((_R, _LK), jnp.int32),
            pltpu.VMEM((_D, _D), jnp.float32),
            pltpu.SMEM((1, 1), jnp.float32),
            pltpu.SMEM((1, 1), jnp.float32),
        ],
        compiler_params=pltpu.CompilerParams(
            dimension_semantics=("arbitrary", "arbitrary")),
        interpret=interpret,
    )


def _ew_body(x_ref, q_ref, o_ref):
    o_ref[...] = x_ref[...] + (q_ref[...] - x_ref[...])


def _make_ew(interpret=False):
    return pl.pallas_call(
        _ew_body,
        grid=(8,),
        in_specs=[
            pl.BlockSpec((1024, _D), lambda i: (i, 0)),
            pl.BlockSpec((1024, _D), lambda i: (i, 0)),
        ],
        out_specs=pl.BlockSpec((1024, _D), lambda i: (i, 0)),
        out_shape=jax.ShapeDtypeStruct((_N, _D), jnp.float32),
        interpret=interpret,
    )


def _sc_gather(table, idx):
    mesh = plsc.VectorSubcoreMesh(core_axis_name="c", subcore_axis_name="s")
    n_workers = 32
    bpw = _N // n_workers

    @functools.partial(
        pl.kernel,
        out_type=jax.ShapeDtypeStruct((_N, _D), jnp.float32),
        mesh=mesh,
        scratch_types=[
            pltpu.VMEM((bpw,), jnp.int32),
            pltpu.VMEM((bpw, _D), jnp.float32),
            pltpu.SemaphoreType.DMA,
        ],
    )
    def gather_kernel(table_hbm, idx_hbm, out_hbm, idx_v, rows_v, sem):
        wid = jax.lax.axis_index("s") * 2 + jax.lax.axis_index("c")
        base = wid * bpw
        pltpu.sync_copy(idx_hbm.at[pl.ds(base, bpw)], idx_v)
        pltpu.async_copy(table_hbm.at[idx_v], rows_v, sem).wait()
        pltpu.sync_copy(rows_v, out_hbm.at[pl.ds(base, bpw)])

    return gather_kernel(table, idx)


def kernel(inputs, embedding_weight):
    input_shape = inputs.shape
    x = inputs.reshape(-1, _D)
    xsq = jnp.sum(x ** 2, axis=1, keepdims=True)
    esq = jnp.sum(embedding_weight ** 2, axis=1)

    idx2d, loss11 = _make_phase1()(2.0 * x, embedding_weight, xsq,
                                   esq.reshape(1, _N))
    q = _sc_gather(embedding_weight, idx2d.reshape(_N))
    quantized_st = _make_ew()(x, q)
    return (quantized_st.reshape(input_shape), loss11[0, 0], idx2d, inputs)


# speedup vs baseline: 1.4935x; 1.0257x over previous
import functools

import jax
import jax.numpy as jnp
from jax.experimental import pallas as pl
from jax.experimental.pallas import tpu as pltpu
from jax.experimental.pallas import tpu_sc as plsc

_N = 8192
_D = 256
_R = 4096
_C = 1024
_NI = _N // _R
_NJ = _N // _C


_RS = 64
_LK = 128


def _phase1_body(x2_ref, e_ref, xsq_ref, esq_ref, idx_ref, loss_ref,
                 rv, rc, gram, sum_min, sum_e4):
    i = pl.program_id(0)
    j = pl.program_id(1)

    @pl.when(j == 0)
    def _():
        rv[...] = jnp.full((_R, _LK), jnp.float32(3.0e38), jnp.float32)
        rc[...] = jnp.zeros((_R, _LK), jnp.int32)

    xe2 = jax.lax.dot_general(x2_ref[...] + x2_ref[...], e_ref[...],
                              (((1,), (1,)), ((), ())),
                              preferred_element_type=jnp.float32)

    for rs in range(_R // _RS):
        rsl = pl.ds(rs * _RS, _RS)
        xsq_s = xsq_ref[rsl, :]
        tv = None
        tc = None
        for c in range(_C // _LK):
            esq_c = esq_ref[:, pl.ds(c * _LK, _LK)]
            t1 = xsq_s + esq_c
            dch = t1 - xe2[rs * _RS:(rs + 1) * _RS, c * _LK:(c + 1) * _LK]
            if tv is None:
                tv = dch
                tc = jnp.zeros((_RS, _LK), jnp.int32)
            else:
                b = dch < tv
                tv = jnp.where(b, dch, tv)
                tc = jnp.where(b, jnp.int32(c), tc)
        old = rv[rsl, :]
        b = tv < old
        rv[rsl, :] = jnp.where(b, tv, old)
        rc[rsl, :] = jnp.where(b, tc + jnp.int32(j * (_C // _LK)), rc[rsl, :])

    @pl.when(i == 0)
    def _():
        g = jax.lax.dot_general(e_ref[...], e_ref[...], (((0,), (0,)), ((), ())),
                                preferred_element_type=jnp.float32)
        esq_v = esq_ref[...]
        s4 = jnp.sum(esq_v * esq_v)

        @pl.when(j == 0)
        def _():
            gram[...] = g
            sum_e4[0, 0] = s4

        @pl.when(j > 0)
        def _():
            gram[...] += g
            sum_e4[0, 0] += s4

    @pl.when(j == _NJ - 1)
    def _():
        lane = jax.lax.broadcasted_iota(jnp.int32, (_RS, _LK), 1)
        acc = jnp.float32(0.0)
        for rs in range(_R // _RS):
            rsl = pl.ds(rs * _RS, _RS)
            v = rv[rsl, :]
            tmin = jnp.min(v, axis=1, keepdims=True)
            col = rc[rsl, :] * _LK + lane
            targ = jnp.min(jnp.where(v == tmin, col, jnp.int32(2 ** 30)),
                           axis=1, keepdims=True)
            idx_ref[rsl, :] = targ
            acc = acc + jnp.sum(tmin)

        @pl.when(i == 0)
        def _():
            sum_min[0, 0] = acc

        @pl.when(i > 0)
        def _():
            sum_min[0, 0] += acc

    @pl.when((i == _NI - 1) & (j == _NJ - 1))
    def _():
        m = gram[...]
        ortho_sq = jnp.sum(m * m) - sum_e4[0, 0]
        ortho = jnp.sqrt(jnp.maximum(ortho_sq, 0.0))
        mse = sum_min[0, 0] / jnp.float32(_N * _D)
        loss_ref[...] = jnp.full((1, 1), mse + 0.25 * mse + 0.09 * ortho,
                                 jnp.float32)


def _make_phase1(interpret=False):
    return pl.pallas_call(
        _phase1_body,
        grid=(_NI, _NJ),
        in_specs=[
            pl.BlockSpec((_R, _D), lambda i, j: (i, 0)),
            pl.BlockSpec((_C, _D), lambda i, j: (j, 0)),
            pl.BlockSpec((_R, 1), lambda i, j: (i, 0)),
            pl.BlockSpec((1, _C), lambda i, j: (0, j)),
        ],
        out_specs=[
            pl.BlockSpec((_R, 1), lambda i, j: (i, 0)),
            pl.BlockSpec((1, 1), lambda i, j: (0, 0)),
        ],
        out_shape=[
            jax.ShapeDtypeStruct((_N, 1), jnp.int32),
            jax.ShapeDtypeStruct((1, 1), jnp.float32),
        ],
        scratch_shapes=[
            pltpu.VMEM((_R, _LK), jnp.float32),
            pltpu.VMEM((_R, _LK), jnp.int32),
            pltpu.VMEM((_D, _D), jnp.float32),
            pltpu.SMEM((1, 1), jnp.float32),
            pltpu.SMEM((1, 1), jnp.float32),
        ],
        compiler_params=pltpu.CompilerParams(
            dimension_semantics=("arbitrary", "arbitrary")),
        interpret=interpret,
    )


def _ew_body(x_ref, q_ref, o_ref):
    o_ref[...] = x_ref[...] + (q_ref[...] - x_ref[...])


def _make_ew(interpret=False):
    return pl.pallas_call(
        _ew_body,
        grid=(8,),
        in_specs=[
            pl.BlockSpec((1024, _D), lambda i: (i, 0)),
            pl.BlockSpec((1024, _D), lambda i: (i, 0)),
        ],
        out_specs=pl.BlockSpec((1024, _D), lambda i: (i, 0)),
        out_shape=jax.ShapeDtypeStruct((_N, _D), jnp.float32),
        interpret=interpret,
    )


def _sc_gather(table, idx):
    mesh = plsc.VectorSubcoreMesh(core_axis_name="c", subcore_axis_name="s")
    n_workers = 32
    bpw = _N // n_workers

    @functools.partial(
        pl.kernel,
        out_type=jax.ShapeDtypeStruct((_N, _D), jnp.float32),
        mesh=mesh,
        scratch_types=[
            pltpu.VMEM((bpw,), jnp.int32),
            pltpu.VMEM((bpw, _D), jnp.float32),
            pltpu.SemaphoreType.DMA,
        ],
    )
    def gather_kernel(table_hbm, idx_hbm, out_hbm, idx_v, rows_v, sem):
        wid = jax.lax.axis_index("s") * 2 + jax.lax.axis_index("c")
        base = wid * bpw
        pltpu.sync_copy(idx_hbm.at[pl.ds(base, bpw)], idx_v)
        pltpu.async_copy(table_hbm.at[idx_v], rows_v, sem).wait()
        pltpu.sync_copy(rows_v, out_hbm.at[pl.ds(base, bpw)])

    return gather_kernel(table, idx)


def kernel(inputs, embedding_weight):
    input_shape = inputs.shape
    x = inputs.reshape(-1, _D)
    xsq = jnp.sum(x ** 2, axis=1, keepdims=True)
    esq = jnp.sum(embedding_weight ** 2, axis=1)

    idx2d, loss11 = _make_phase1()(x, embedding_weight, xsq,
                                   esq.reshape(1, _N))
    q = _sc_gather(embedding_weight, idx2d.reshape(_N))
    quantized_st = _make_ew()(x, q)
    return (quantized_st.reshape(input_shape), loss11[0, 0], idx2d, inputs)


# speedup vs baseline: 1.5598x; 1.0443x over previous
import functools

import jax
import jax.numpy as jnp
from jax.experimental import pallas as pl
from jax.experimental.pallas import tpu as pltpu
from jax.experimental.pallas import tpu_sc as plsc

_N = 8192
_D = 256
_R = 4096
_C = 1024
_NI = _N // _R
_NJ = _N // _C


_RS = 128
_LK = 128


def _phase1_body(x2_ref, e_ref, xsq_ref, esq_ref, idx_ref, loss_ref,
                 rv, rc, gram, sum_min, sum_e4):
    i = pl.program_id(0)
    j = pl.program_id(1)

    @pl.when(j == 0)
    def _():
        rv[...] = jnp.full((_R, _LK), jnp.float32(3.0e38), jnp.float32)
        rc[...] = jnp.zeros((_R, _LK), jnp.int32)

    xe2 = jax.lax.dot_general(x2_ref[...] + x2_ref[...], e_ref[...],
                              (((1,), (1,)), ((), ())),
                              preferred_element_type=jnp.float32)

    for rs in range(_R // _RS):
        rsl = pl.ds(rs * _RS, _RS)
        xsq_s = xsq_ref[rsl, :]
        tv = None
        tc = None
        for c in range(_C // _LK):
            esq_c = esq_ref[:, pl.ds(c * _LK, _LK)]
            t1 = xsq_s + esq_c
            dch = t1 - xe2[rs * _RS:(rs + 1) * _RS, c * _LK:(c + 1) * _LK]
            if tv is None:
                tv = dch
                tc = jnp.zeros((_RS, _LK), jnp.int32)
            else:
                b = dch < tv
                tv = jnp.where(b, dch, tv)
                tc = jnp.where(b, jnp.int32(c), tc)
        old = rv[rsl, :]
        b = tv < old
        rv[rsl, :] = jnp.where(b, tv, old)
        rc[rsl, :] = jnp.where(b, tc + jnp.int32(j * (_C // _LK)), rc[rsl, :])

    @pl.when(i == 0)
    def _():
        g = jax.lax.dot_general(e_ref[...], e_ref[...], (((0,), (0,)), ((), ())),
                                preferred_element_type=jnp.float32)
        esq_v = esq_ref[...]
        s4 = jnp.sum(esq_v * esq_v)

        @pl.when(j == 0)
        def _():
            gram[...] = g
            sum_e4[0, 0] = s4

        @pl.when(j > 0)
        def _():
            gram[...] += g
            sum_e4[0, 0] += s4

    @pl.when(j == _NJ - 1)
    def _():
        lane = jax.lax.broadcasted_iota(jnp.int32, (_RS, _LK), 1)
        acc = jnp.float32(0.0)
        for rs in range(_R // _RS):
            rsl = pl.ds(rs * _RS, _RS)
            v = rv[rsl, :]
            tmin = jnp.min(v, axis=1, keepdims=True)
            col = rc[rsl, :] * _LK + lane
            targ = jnp.min(jnp.where(v == tmin, col, jnp.int32(2 ** 30)),
                           axis=1, keepdims=True)
            idx_ref[rsl, :] = targ
            acc = acc + jnp.sum(tmin)

        @pl.when(i == 0)
        def _():
            sum_min[0, 0] = acc

        @pl.when(i > 0)
        def _():
            sum_min[0, 0] += acc

    @pl.when((i == _NI - 1) & (j == _NJ - 1))
    def _():
        m = gram[...]
        ortho_sq = jnp.sum(m * m) - sum_e4[0, 0]
        ortho = jnp.sqrt(jnp.maximum(ortho_sq, 0.0))
        mse = sum_min[0, 0] / jnp.float32(_N * _D)
        loss_ref[...] = jnp.full((1, 1), mse + 0.25 * mse + 0.09 * ortho,
                                 jnp.float32)


def _make_phase1(interpret=False):
    return pl.pallas_call(
        _phase1_body,
        grid=(_NI, _NJ),
        in_specs=[
            pl.BlockSpec((_R, _D), lambda i, j: (i, 0)),
            pl.BlockSpec((_C, _D), lambda i, j: (j, 0)),
            pl.BlockSpec((_R, 1), lambda i, j: (i, 0)),
            pl.BlockSpec((1, _C), lambda i, j: (0, j)),
        ],
        out_specs=[
            pl.BlockSpec((_R, 1), lambda i, j: (i, 0)),
            pl.BlockSpec((1, 1), lambda i, j: (0, 0)),
        ],
        out_shape=[
            jax.ShapeDtypeStruct((_N, 1), jnp.int32),
            jax.ShapeDtypeStruct((1, 1), jnp.float32),
        ],
        scratch_shapes=[
            pltpu.VMEM((_R, _LK), jnp.float32),
            pltpu.VMEM((_R, _LK), jnp.int32),
            pltpu.VMEM((_D, _D), jnp.float32),
            pltpu.SMEM((1, 1), jnp.float32),
            pltpu.SMEM((1, 1), jnp.float32),
        ],
        compiler_params=pltpu.CompilerParams(
            dimension_semantics=("arbitrary", "arbitrary")),
        interpret=interpret,
    )


def _ew_body(x_ref, q_ref, o_ref):
    o_ref[...] = x_ref[...] + (q_ref[...] - x_ref[...])


def _make_ew(interpret=False):
    return pl.pallas_call(
        _ew_body,
        grid=(8,),
        in_specs=[
            pl.BlockSpec((1024, _D), lambda i: (i, 0)),
            pl.BlockSpec((1024, _D), lambda i: (i, 0)),
        ],
        out_specs=pl.BlockSpec((1024, _D), lambda i: (i, 0)),
        out_shape=jax.ShapeDtypeStruct((_N, _D), jnp.float32),
        interpret=interpret,
    )


def _sc_gather(table, idx):
    mesh = plsc.VectorSubcoreMesh(core_axis_name="c", subcore_axis_name="s")
    n_workers = 32
    bpw = _N // n_workers

    @functools.partial(
        pl.kernel,
        out_type=jax.ShapeDtypeStruct((_N, _D), jnp.float32),
        mesh=mesh,
        scratch_types=[
            pltpu.VMEM((bpw,), jnp.int32),
            pltpu.VMEM((bpw, _D), jnp.float32),
            pltpu.SemaphoreType.DMA,
        ],
    )
    def gather_kernel(table_hbm, idx_hbm, out_hbm, idx_v, rows_v, sem):
        wid = jax.lax.axis_index("s") * 2 + jax.lax.axis_index("c")
        base = wid * bpw
        pltpu.sync_copy(idx_hbm.at[pl.ds(base, bpw)], idx_v)
        pltpu.async_copy(table_hbm.at[idx_v], rows_v, sem).wait()
        pltpu.sync_copy(rows_v, out_hbm.at[pl.ds(base, bpw)])

    return gather_kernel(table, idx)


def kernel(inputs, embedding_weight):
    input_shape = inputs.shape
    x = inputs.reshape(-1, _D)
    xsq = jnp.sum(x ** 2, axis=1, keepdims=True)
    esq = jnp.sum(embedding_weight ** 2, axis=1)

    idx2d, loss11 = _make_phase1()(x, embedding_weight, xsq,
                                   esq.reshape(1, _N))
    q = _sc_gather(embedding_weight, idx2d.reshape(_N))
    quantized_st = _make_ew()(x, q)
    return (quantized_st.reshape(input_shape), loss11[0, 0], idx2d, inputs)


# speedup vs baseline: 1.6020x; 1.0271x over previous
import functools

import jax
import jax.numpy as jnp
from jax.experimental import pallas as pl
from jax.experimental.pallas import tpu as pltpu
from jax.experimental.pallas import tpu_sc as plsc

_N = 8192
_D = 256
_R = 4096
_C = 1024
_NI = _N // _R
_NJ = _N // _C


_RS = 256
_LK = 128


def _phase1_body(x2_ref, e_ref, xsq_ref, esq_ref, idx_ref, loss_ref,
                 rv, rc, gram, sum_min, sum_e4):
    i = pl.program_id(0)
    j = pl.program_id(1)

    @pl.when(j == 0)
    def _():
        rv[...] = jnp.full((_R, _LK), jnp.float32(3.0e38), jnp.float32)
        rc[...] = jnp.zeros((_R, _LK), jnp.int32)

    xe2 = jax.lax.dot_general(x2_ref[...] + x2_ref[...], e_ref[...],
                              (((1,), (1,)), ((), ())),
                              preferred_element_type=jnp.float32)

    for rs in range(_R // _RS):
        rsl = pl.ds(rs * _RS, _RS)
        xsq_s = xsq_ref[rsl, :]
        tv = None
        tc = None
        for c in range(_C // _LK):
            esq_c = esq_ref[:, pl.ds(c * _LK, _LK)]
            t1 = xsq_s + esq_c
            dch = t1 - xe2[rs * _RS:(rs + 1) * _RS, c * _LK:(c + 1) * _LK]
            if tv is None:
                tv = dch
                tc = jnp.zeros((_RS, _LK), jnp.int32)
            else:
                b = dch < tv
                tv = jnp.where(b, dch, tv)
                tc = jnp.where(b, jnp.int32(c), tc)
        old = rv[rsl, :]
        b = tv < old
        rv[rsl, :] = jnp.where(b, tv, old)
        rc[rsl, :] = jnp.where(b, tc + jnp.int32(j * (_C // _LK)), rc[rsl, :])

    @pl.when(i == 0)
    def _():
        g = jax.lax.dot_general(e_ref[...], e_ref[...], (((0,), (0,)), ((), ())),
                                preferred_element_type=jnp.float32)
        esq_v = esq_ref[...]
        s4 = jnp.sum(esq_v * esq_v)

        @pl.when(j == 0)
        def _():
            gram[...] = g
            sum_e4[0, 0] = s4

        @pl.when(j > 0)
        def _():
            gram[...] += g
            sum_e4[0, 0] += s4

    @pl.when(j == _NJ - 1)
    def _():
        lane = jax.lax.broadcasted_iota(jnp.int32, (_RS, _LK), 1)
        acc = jnp.float32(0.0)
        for rs in range(_R // _RS):
            rsl = pl.ds(rs * _RS, _RS)
            v = rv[rsl, :]
            tmin = jnp.min(v, axis=1, keepdims=True)
            col = rc[rsl, :] * _LK + lane
            targ = jnp.min(jnp.where(v == tmin, col, jnp.int32(2 ** 30)),
                           axis=1, keepdims=True)
            idx_ref[rsl, :] = targ
            acc = acc + jnp.sum(tmin)

        @pl.when(i == 0)
        def _():
            sum_min[0, 0] = acc

        @pl.when(i > 0)
        def _():
            sum_min[0, 0] += acc

    @pl.when((i == _NI - 1) & (j == _NJ - 1))
    def _():
        m = gram[...]
        ortho_sq = jnp.sum(m * m) - sum_e4[0, 0]
        ortho = jnp.sqrt(jnp.maximum(ortho_sq, 0.0))
        mse = sum_min[0, 0] / jnp.float32(_N * _D)
        loss_ref[...] = jnp.full((1, 1), mse + 0.25 * mse + 0.09 * ortho,
                                 jnp.float32)


def _make_phase1(interpret=False):
    return pl.pallas_call(
        _phase1_body,
        grid=(_NI, _NJ),
        in_specs=[
            pl.BlockSpec((_R, _D), lambda i, j: (i, 0)),
            pl.BlockSpec((_C, _D), lambda i, j: (j, 0)),
            pl.BlockSpec((_R, 1), lambda i, j: (i, 0)),
            pl.BlockSpec((1, _C), lambda i, j: (0, j)),
        ],
        out_specs=[
            pl.BlockSpec((_R, 1), lambda i, j: (i, 0)),
            pl.BlockSpec((1, 1), lambda i, j: (0, 0)),
        ],
        out_shape=[
            jax.ShapeDtypeStruct((_N, 1), jnp.int32),
            jax.ShapeDtypeStruct((1, 1), jnp.float32),
        ],
        scratch_shapes=[
            pltpu.VMEM((_R, _LK), jnp.float32),
            pltpu.VMEM((_R, _LK), jnp.int32),
            pltpu.VMEM((_D, _D), jnp.float32),
            pltpu.SMEM((1, 1), jnp.float32),
            pltpu.SMEM((1, 1), jnp.float32),
        ],
        compiler_params=pltpu.CompilerParams(
            dimension_semantics=("arbitrary", "arbitrary")),
        interpret=interpret,
    )


def _ew_body(x_ref, q_ref, o_ref):
    o_ref[...] = x_ref[...] + (q_ref[...] - x_ref[...])


def _make_ew(interpret=False):
    return pl.pallas_call(
        _ew_body,
        grid=(8,),
        in_specs=[
            pl.BlockSpec((1024, _D), lambda i: (i, 0)),
            pl.BlockSpec((1024, _D), lambda i: (i, 0)),
        ],
        out_specs=pl.BlockSpec((1024, _D), lambda i: (i, 0)),
        out_shape=jax.ShapeDtypeStruct((_N, _D), jnp.float32),
        interpret=interpret,
    )


def _sc_gather(table, idx):
    mesh = plsc.VectorSubcoreMesh(core_axis_name="c", subcore_axis_name="s")
    n_workers = 32
    bpw = _N // n_workers

    @functools.partial(
        pl.kernel,
        out_type=jax.ShapeDtypeStruct((_N, _D), jnp.float32),
        mesh=mesh,
        scratch_types=[
            pltpu.VMEM((bpw,), jnp.int32),
            pltpu.VMEM((bpw, _D), jnp.float32),
            pltpu.SemaphoreType.DMA,
        ],
    )
    def gather_kernel(table_hbm, idx_hbm, out_hbm, idx_v, rows_v, sem):
        wid = jax.lax.axis_index("s") * 2 + jax.lax.axis_index("c")
        base = wid * bpw
        pltpu.sync_copy(idx_hbm.at[pl.ds(base, bpw)], idx_v)
        pltpu.async_copy(table_hbm.at[idx_v], rows_v, sem).wait()
        pltpu.sync_copy(rows_v, out_hbm.at[pl.ds(base, bpw)])

    return gather_kernel(table, idx)


def kernel(inputs, embedding_weight):
    input_shape = inputs.shape
    x = inputs.reshape(-1, _D)
    xsq = jnp.sum(x ** 2, axis=1, keepdims=True)
    esq = jnp.sum(embedding_weight ** 2, axis=1)

    idx2d, loss11 = _make_phase1()(x, embedding_weight, xsq,
                                   esq.reshape(1, _N))
    q = _sc_gather(embedding_weight, idx2d.reshape(_N))
    quantized_st = _make_ew()(x, q)
    return (quantized_st.reshape(input_shape), loss11[0, 0], idx2d, inputs)


# speedup vs baseline: 1.6397x; 1.0235x over previous
import functools

import jax
import jax.numpy as jnp
from jax.experimental import pallas as pl
from jax.experimental.pallas import tpu as pltpu
from jax.experimental.pallas import tpu_sc as plsc

_N = 8192
_D = 256
_R = 4096
_C = 1024
_NI = _N // _R
_NJ = _N // _C


_RS = 512
_LK = 128


def _phase1_body(x2_ref, e_ref, xsq_ref, esq_ref, idx_ref, loss_ref,
                 rv, rc, gram, sum_min, sum_e4):
    i = pl.program_id(0)
    j = pl.program_id(1)

    @pl.when(j == 0)
    def _():
        rv[...] = jnp.full((_R, _LK), jnp.float32(3.0e38), jnp.float32)
        rc[...] = jnp.zeros((_R, _LK), jnp.int32)

    xe2 = jax.lax.dot_general(x2_ref[...] + x2_ref[...], e_ref[...],
                              (((1,), (1,)), ((), ())),
                              preferred_element_type=jnp.float32)

    for rs in range(_R // _RS):
        rsl = pl.ds(rs * _RS, _RS)
        xsq_s = xsq_ref[rsl, :]
        tv = None
        tc = None
        for c in range(_C // _LK):
            esq_c = esq_ref[:, pl.ds(c * _LK, _LK)]
            t1 = xsq_s + esq_c
            dch = t1 - xe2[rs * _RS:(rs + 1) * _RS, c * _LK:(c + 1) * _LK]
            if tv is None:
                tv = dch
                tc = jnp.zeros((_RS, _LK), jnp.int32)
            else:
                b = dch < tv
                tv = jnp.where(b, dch, tv)
                tc = jnp.where(b, jnp.int32(c), tc)
        old = rv[rsl, :]
        b = tv < old
        rv[rsl, :] = jnp.where(b, tv, old)
        rc[rsl, :] = jnp.where(b, tc + jnp.int32(j * (_C // _LK)), rc[rsl, :])

    @pl.when(i == 0)
    def _():
        g = jax.lax.dot_general(e_ref[...], e_ref[...], (((0,), (0,)), ((), ())),
                                preferred_element_type=jnp.float32)
        esq_v = esq_ref[...]
        s4 = jnp.sum(esq_v * esq_v)

        @pl.when(j == 0)
        def _():
            gram[...] = g
            sum_e4[0, 0] = s4

        @pl.when(j > 0)
        def _():
            gram[...] += g
            sum_e4[0, 0] += s4

    @pl.when(j == _NJ - 1)
    def _():
        lane = jax.lax.broadcasted_iota(jnp.int32, (_RS, _LK), 1)
        acc = jnp.float32(0.0)
        for rs in range(_R // _RS):
            rsl = pl.ds(rs * _RS, _RS)
            v = rv[rsl, :]
            tmin = jnp.min(v, axis=1, keepdims=True)
            col = rc[rsl, :] * _LK + lane
            targ = jnp.min(jnp.where(v == tmin, col, jnp.int32(2 ** 30)),
                           axis=1, keepdims=True)
            idx_ref[rsl, :] = targ
            acc = acc + jnp.sum(tmin)

        @pl.when(i == 0)
        def _():
            sum_min[0, 0] = acc

        @pl.when(i > 0)
        def _():
            sum_min[0, 0] += acc

    @pl.when((i == _NI - 1) & (j == _NJ - 1))
    def _():
        m = gram[...]
        ortho_sq = jnp.sum(m * m) - sum_e4[0, 0]
        ortho = jnp.sqrt(jnp.maximum(ortho_sq, 0.0))
        mse = sum_min[0, 0] / jnp.float32(_N * _D)
        loss_ref[...] = jnp.full((1, 1), mse + 0.25 * mse + 0.09 * ortho,
                                 jnp.float32)


def _make_phase1(interpret=False):
    return pl.pallas_call(
        _phase1_body,
        grid=(_NI, _NJ),
        in_specs=[
            pl.BlockSpec((_R, _D), lambda i, j: (i, 0)),
            pl.BlockSpec((_C, _D), lambda i, j: (j, 0)),
            pl.BlockSpec((_R, 1), lambda i, j: (i, 0)),
            pl.BlockSpec((1, _C), lambda i, j: (0, j)),
        ],
        out_specs=[
            pl.BlockSpec((_R, 1), lambda i, j: (i, 0)),
            pl.BlockSpec((1, 1), lambda i, j: (0, 0)),
        ],
        out_shape=[
            jax.ShapeDtypeStruct((_N, 1), jnp.int32),
            jax.ShapeDtypeStruct((1, 1), jnp.float32),
        ],
        scratch_shapes=[
            pltpu.VMEM((_R, _LK), jnp.float32),
            pltpu.VMEM((_R, _LK), jnp.int32),
            pltpu.VMEM((_D, _D), jnp.float32),
            pltpu.SMEM((1, 1), jnp.float32),
            pltpu.SMEM((1, 1), jnp.float32),
        ],
        compiler_params=pltpu.CompilerParams(
            dimension_semantics=("arbitrary", "arbitrary")),
        interpret=interpret,
    )


def _ew_body(x_ref, q_ref, o_ref):
    o_ref[...] = x_ref[...] + (q_ref[...] - x_ref[...])


def _make_ew(interpret=False):
    return pl.pallas_call(
        _ew_body,
        grid=(8,),
        in_specs=[
            pl.BlockSpec((1024, _D), lambda i: (i, 0)),
            pl.BlockSpec((1024, _D), lambda i: (i, 0)),
        ],
        out_specs=pl.BlockSpec((1024, _D), lambda i: (i, 0)),
        out_shape=jax.ShapeDtypeStruct((_N, _D), jnp.float32),
        interpret=interpret,
    )


def _sc_gather(table, idx):
    mesh = plsc.VectorSubcoreMesh(core_axis_name="c", subcore_axis_name="s")
    n_workers = 32
    bpw = _N // n_workers

    @functools.partial(
        pl.kernel,
        out_type=jax.ShapeDtypeStruct((_N, _D), jnp.float32),
        mesh=mesh,
        scratch_types=[
            pltpu.VMEM((bpw,), jnp.int32),
            pltpu.VMEM((bpw, _D), jnp.float32),
            pltpu.SemaphoreType.DMA,
        ],
    )
    def gather_kernel(table_hbm, idx_hbm, out_hbm, idx_v, rows_v, sem):
        wid = jax.lax.axis_index("s") * 2 + jax.lax.axis_index("c")
        base = wid * bpw
        pltpu.sync_copy(idx_hbm.at[pl.ds(base, bpw)], idx_v)
        pltpu.async_copy(table_hbm.at[idx_v], rows_v, sem).wait()
        pltpu.sync_copy(rows_v, out_hbm.at[pl.ds(base, bpw)])

    return gather_kernel(table, idx)


def kernel(inputs, embedding_weight):
    input_shape = inputs.shape
    x = inputs.reshape(-1, _D)
    xsq = jnp.sum(x ** 2, axis=1, keepdims=True)
    esq = jnp.sum(embedding_weight ** 2, axis=1)

    idx2d, loss11 = _make_phase1()(x, embedding_weight, xsq,
                                   esq.reshape(1, _N))
    q = _sc_gather(embedding_weight, idx2d.reshape(_N))
    quantized_st = _make_ew()(x, q)
    return (quantized_st.reshape(input_shape), loss11[0, 0], idx2d, inputs)


# speedup vs baseline: 1.6440x; 1.0026x over previous
import functools

import jax
import jax.numpy as jnp
from jax.experimental import pallas as pl
from jax.experimental.pallas import tpu as pltpu
from jax.experimental.pallas import tpu_sc as plsc

_N = 8192
_D = 256
_R = 4096
_C = 1024
_NI = _N // _R
_NJ = _N // _C


_RS = 1024
_LK = 128


def _phase1_body(x2_ref, e_ref, xsq_ref, esq_ref, idx_ref, loss_ref,
                 rv, rc, gram, sum_min, sum_e4):
    i = pl.program_id(0)
    j = pl.program_id(1)

    @pl.when(j == 0)
    def _():
        rv[...] = jnp.full((_R, _LK), jnp.float32(3.0e38), jnp.float32)
        rc[...] = jnp.zeros((_R, _LK), jnp.int32)

    xe2 = jax.lax.dot_general(x2_ref[...] + x2_ref[...], e_ref[...],
                              (((1,), (1,)), ((), ())),
                              preferred_element_type=jnp.float32)

    for rs in range(_R // _RS):
        rsl = pl.ds(rs * _RS, _RS)
        xsq_s = xsq_ref[rsl, :]
        tv = None
        tc = None
        for c in range(_C // _LK):
            esq_c = esq_ref[:, pl.ds(c * _LK, _LK)]
            t1 = xsq_s + esq_c
            dch = t1 - xe2[rs * _RS:(rs + 1) * _RS, c * _LK:(c + 1) * _LK]
            if tv is None:
                tv = dch
                tc = jnp.zeros((_RS, _LK), jnp.int32)
            else:
                b = dch < tv
                tv = jnp.where(b, dch, tv)
                tc = jnp.where(b, jnp.int32(c), tc)
        old = rv[rsl, :]
        b = tv < old
        rv[rsl, :] = jnp.where(b, tv, old)
        rc[rsl, :] = jnp.where(b, tc + jnp.int32(j * (_C // _LK)), rc[rsl, :])

    @pl.when(i == 0)
    def _():
        g = jax.lax.dot_general(e_ref[...], e_ref[...], (((0,), (0,)), ((), ())),
                                preferred_element_type=jnp.float32)
        esq_v = esq_ref[...]
        s4 = jnp.sum(esq_v * esq_v)

        @pl.when(j == 0)
        def _():
            gram[...] = g
            sum_e4[0, 0] = s4

        @pl.when(j > 0)
        def _():
            gram[...] += g
            sum_e4[0, 0] += s4

    @pl.when(j == _NJ - 1)
    def _():
        lane = jax.lax.broadcasted_iota(jnp.int32, (_RS, _LK), 1)
        acc = jnp.float32(0.0)
        for rs in range(_R // _RS):
            rsl = pl.ds(rs * _RS, _RS)
            v = rv[rsl, :]
            tmin = jnp.min(v, axis=1, keepdims=True)
            col = rc[rsl, :] * _LK + lane
            targ = jnp.min(jnp.where(v == tmin, col, jnp.int32(2 ** 30)),
                           axis=1, keepdims=True)
            idx_ref[rsl, :] = targ
            acc = acc + jnp.sum(tmin)

        @pl.when(i == 0)
        def _():
            sum_min[0, 0] = acc

        @pl.when(i > 0)
        def _():
            sum_min[0, 0] += acc

    @pl.when((i == _NI - 1) & (j == _NJ - 1))
    def _():
        m = gram[...]
        ortho_sq = jnp.sum(m * m) - sum_e4[0, 0]
        ortho = jnp.sqrt(jnp.maximum(ortho_sq, 0.0))
        mse = sum_min[0, 0] / jnp.float32(_N * _D)
        loss_ref[...] = jnp.full((1, 1), mse + 0.25 * mse + 0.09 * ortho,
                                 jnp.float32)


def _make_phase1(interpret=False):
    return pl.pallas_call(
        _phase1_body,
        grid=(_NI, _NJ),
        in_specs=[
            pl.BlockSpec((_R, _D), lambda i, j: (i, 0)),
            pl.BlockSpec((_C, _D), lambda i, j: (j, 0)),
            pl.BlockSpec((_R, 1), lambda i, j: (i, 0)),
            pl.BlockSpec((1, _C), lambda i, j: (0, j)),
        ],
        out_specs=[
            pl.BlockSpec((_R, 1), lambda i, j: (i, 0)),
            pl.BlockSpec((1, 1), lambda i, j: (0, 0)),
        ],
        out_shape=[
            jax.ShapeDtypeStruct((_N, 1), jnp.int32),
            jax.ShapeDtypeStruct((1, 1), jnp.float32),
        ],
        scratch_shapes=[
            pltpu.VMEM((_R, _LK), jnp.float32),
            pltpu.VMEM((_R, _LK), jnp.int32),
            pltpu.VMEM((_D, _D), jnp.float32),
            pltpu.SMEM((1, 1), jnp.float32),
            pltpu.SMEM((1, 1), jnp.float32),
        ],
        compiler_params=pltpu.CompilerParams(
            dimension_semantics=("arbitrary", "arbitrary")),
        interpret=interpret,
    )


def _ew_body(x_ref, q_ref, o_ref):
    o_ref[...] = x_ref[...] + (q_ref[...] - x_ref[...])


def _make_ew(interpret=False):
    return pl.pallas_call(
        _ew_body,
        grid=(8,),
        in_specs=[
            pl.BlockSpec((1024, _D), lambda i: (i, 0)),
            pl.BlockSpec((1024, _D), lambda i: (i, 0)),
        ],
        out_specs=pl.BlockSpec((1024, _D), lambda i: (i, 0)),
        out_shape=jax.ShapeDtypeStruct((_N, _D), jnp.float32),
        interpret=interpret,
    )


def _sc_gather(table, idx):
    mesh = plsc.VectorSubcoreMesh(core_axis_name="c", subcore_axis_name="s")
    n_workers = 32
    bpw = _N // n_workers

    @functools.partial(
        pl.kernel,
        out_type=jax.ShapeDtypeStruct((_N, _D), jnp.float32),
        mesh=mesh,
        scratch_types=[
            pltpu.VMEM((bpw,), jnp.int32),
            pltpu.VMEM((bpw, _D), jnp.float32),
            pltpu.SemaphoreType.DMA,
        ],
    )
    def gather_kernel(table_hbm, idx_hbm, out_hbm, idx_v, rows_v, sem):
        wid = jax.lax.axis_index("s") * 2 + jax.lax.axis_index("c")
        base = wid * bpw
        pltpu.sync_copy(idx_hbm.at[pl.ds(base, bpw)], idx_v)
        pltpu.async_copy(table_hbm.at[idx_v], rows_v, sem).wait()
        pltpu.sync_copy(rows_v, out_hbm.at[pl.ds(base, bpw)])

    return gather_kernel(table, idx)


def kernel(inputs, embedding_weight):
    input_shape = inputs.shape
    x = inputs.reshape(-1, _D)
    xsq = jnp.sum(x ** 2, axis=1, keepdims=True)
    esq = jnp.sum(embedding_weight ** 2, axis=1)

    idx2d, loss11 = _make_phase1()(x, embedding_weight, xsq,
                                   esq.reshape(1, _N))
    q = _sc_gather(embedding_weight, idx2d.reshape(_N))
    quantized_st = _make_ew()(x, q)
    return (quantized_st.reshape(input_shape), loss11[0, 0], idx2d, inputs)


# speedup vs baseline: 1.6539x; 1.0060x over previous
import functools

import jax
import jax.numpy as jnp
from jax.experimental import pallas as pl
from jax.experimental.pallas import tpu as pltpu
from jax.experimental.pallas import tpu_sc as plsc

_N = 8192
_D = 256
_R = 4096
_C = 1024
_NI = _N // _R
_NJ = _N // _C


_RS = 4096
_LK = 128


def _phase1_body(x2_ref, e_ref, xsq_ref, esq_ref, idx_ref, loss_ref,
                 rv, rc, gram, sum_min, sum_e4):
    i = pl.program_id(0)
    j = pl.program_id(1)

    @pl.when(j == 0)
    def _():
        rv[...] = jnp.full((_R, _LK), jnp.float32(3.0e38), jnp.float32)
        rc[...] = jnp.zeros((_R, _LK), jnp.int32)

    xe2 = jax.lax.dot_general(x2_ref[...] + x2_ref[...], e_ref[...],
                              (((1,), (1,)), ((), ())),
                              preferred_element_type=jnp.float32)

    for rs in range(_R // _RS):
        rsl = pl.ds(rs * _RS, _RS)
        xsq_s = xsq_ref[rsl, :]
        tv = None
        tc = None
        for c in range(_C // _LK):
            esq_c = esq_ref[:, pl.ds(c * _LK, _LK)]
            t1 = xsq_s + esq_c
            dch = t1 - xe2[rs * _RS:(rs + 1) * _RS, c * _LK:(c + 1) * _LK]
            if tv is None:
                tv = dch
                tc = jnp.zeros((_RS, _LK), jnp.int32)
            else:
                b = dch < tv
                tv = jnp.where(b, dch, tv)
                tc = jnp.where(b, jnp.int32(c), tc)
        old = rv[rsl, :]
        b = tv < old
        rv[rsl, :] = jnp.where(b, tv, old)
        rc[rsl, :] = jnp.where(b, tc + jnp.int32(j * (_C // _LK)), rc[rsl, :])

    @pl.when(i == 0)
    def _():
        g = jax.lax.dot_general(e_ref[...], e_ref[...], (((0,), (0,)), ((), ())),
                                preferred_element_type=jnp.float32)
        esq_v = esq_ref[...]
        s4 = jnp.sum(esq_v * esq_v)

        @pl.when(j == 0)
        def _():
            gram[...] = g
            sum_e4[0, 0] = s4

        @pl.when(j > 0)
        def _():
            gram[...] += g
            sum_e4[0, 0] += s4

    @pl.when(j == _NJ - 1)
    def _():
        lane = jax.lax.broadcasted_iota(jnp.int32, (_RS, _LK), 1)
        acc = jnp.float32(0.0)
        for rs in range(_R // _RS):
            rsl = pl.ds(rs * _RS, _RS)
            v = rv[rsl, :]
            tmin = jnp.min(v, axis=1, keepdims=True)
            col = rc[rsl, :] * _LK + lane
            targ = jnp.min(jnp.where(v == tmin, col, jnp.int32(2 ** 30)),
                           axis=1, keepdims=True)
            idx_ref[rsl, :] = targ
            acc = acc + jnp.sum(tmin)

        @pl.when(i == 0)
        def _():
            sum_min[0, 0] = acc

        @pl.when(i > 0)
        def _():
            sum_min[0, 0] += acc

    @pl.when((i == _NI - 1) & (j == _NJ - 1))
    def _():
        m = gram[...]
        ortho_sq = jnp.sum(m * m) - sum_e4[0, 0]
        ortho = jnp.sqrt(jnp.maximum(ortho_sq, 0.0))
        mse = sum_min[0, 0] / jnp.float32(_N * _D)
        loss_ref[...] = jnp.full((1, 1), mse + 0.25 * mse + 0.09 * ortho,
                                 jnp.float32)


def _make_phase1(interpret=False):
    return pl.pallas_call(
        _phase1_body,
        grid=(_NI, _NJ),
        in_specs=[
            pl.BlockSpec((_R, _D), lambda i, j: (i, 0)),
            pl.BlockSpec((_C, _D), lambda i, j: (j, 0)),
            pl.BlockSpec((_R, 1), lambda i, j: (i, 0)),
            pl.BlockSpec((1, _C), lambda i, j: (0, j)),
        ],
        out_specs=[
            pl.BlockSpec((_R, 1), lambda i, j: (i, 0)),
            pl.BlockSpec((1, 1), lambda i, j: (0, 0)),
        ],
        out_shape=[
            jax.ShapeDtypeStruct((_N, 1), jnp.int32),
            jax.ShapeDtypeStruct((1, 1), jnp.float32),
        ],
        scratch_shapes=[
            pltpu.VMEM((_R, _LK), jnp.float32),
            pltpu.VMEM((_R, _LK), jnp.int32),
            pltpu.VMEM((_D, _D), jnp.float32),
            pltpu.SMEM((1, 1), jnp.float32),
            pltpu.SMEM((1, 1), jnp.float32),
        ],
        compiler_params=pltpu.CompilerParams(
            dimension_semantics=("arbitrary", "arbitrary")),
        interpret=interpret,
    )


def _ew_body(x_ref, q_ref, o_ref):
    o_ref[...] = x_ref[...] + (q_ref[...] - x_ref[...])


def _make_ew(interpret=False):
    return pl.pallas_call(
        _ew_body,
        grid=(8,),
        in_specs=[
            pl.BlockSpec((1024, _D), lambda i: (i, 0)),
            pl.BlockSpec((1024, _D), lambda i: (i, 0)),
        ],
        out_specs=pl.BlockSpec((1024, _D), lambda i: (i, 0)),
        out_shape=jax.ShapeDtypeStruct((_N, _D), jnp.float32),
        interpret=interpret,
    )


def _sc_gather(table, idx):
    mesh = plsc.VectorSubcoreMesh(core_axis_name="c", subcore_axis_name="s")
    n_workers = 32
    bpw = _N // n_workers

    @functools.partial(
        pl.kernel,
        out_type=jax.ShapeDtypeStruct((_N, _D), jnp.float32),
        mesh=mesh,
        scratch_types=[
            pltpu.VMEM((bpw,), jnp.int32),
            pltpu.VMEM((bpw, _D), jnp.float32),
            pltpu.SemaphoreType.DMA,
        ],
    )
    def gather_kernel(table_hbm, idx_hbm, out_hbm, idx_v, rows_v, sem):
        wid = jax.lax.axis_index("s") * 2 + jax.lax.axis_index("c")
        base = wid * bpw
        pltpu.sync_copy(idx_hbm.at[pl.ds(base, bpw)], idx_v)
        pltpu.async_copy(table_hbm.at[idx_v], rows_v, sem).wait()
        pltpu.sync_copy(rows_v, out_hbm.at[pl.ds(base, bpw)])

    return gather_kernel(table, idx)


def kernel(inputs, embedding_weight):
    input_shape = inputs.shape
    x = inputs.reshape(-1, _D)
    xsq = jnp.sum(x ** 2, axis=1, keepdims=True)
    esq = jnp.sum(embedding_weight ** 2, axis=1)

    idx2d, loss11 = _make_phase1()(x, embedding_weight, xsq,
                                   esq.reshape(1, _N))
    q = _sc_gather(embedding_weight, idx2d.reshape(_N))
    quantized_st = _make_ew()(x, q)
    return (quantized_st.reshape(input_shape), loss11[0, 0], idx2d, inputs)
